# Initial kernel scaffold; baseline (speedup 1.0000x reference)
import jax
import jax.numpy as jnp
from jax.experimental import pallas as pl


def kernel(batch, corrupt_batch, entity_embedding, relation_embedding):
    raise NotImplementedError("write your pallas kernel here")


# speedup vs baseline: 1.4801x; 1.4801x over previous
import functools

import jax
import jax.numpy as jnp
from jax import lax
from jax.experimental import pallas as pl
from jax.experimental.pallas import tpu as pltpu
from jax.experimental.pallas import tpu_sc as plsc

EMBED = 64
TOTAL = 2048
NUM_CORES = 2
NUM_SUBCORES = 16
NW = NUM_CORES * NUM_SUBCORES
B_PER_W = TOTAL // NW


def _score_body(ent_hbm, rel_hbm, h_hbm, r_hbm, tt_hbm, t_hbm, out_hbm,
                hidx_v, ridx_v, ttidx_v, tidx_v,
                eh_v, rr_v, rtt_v, et_v, sums_v, out_v,
                sem0, sem1, sem2, sem3):
    wid = lax.axis_index("s") * NUM_CORES + lax.axis_index("c")
    base = wid * B_PER_W

    pltpu.sync_copy(h_hbm.at[pl.ds(base, B_PER_W)], hidx_v)
    pltpu.sync_copy(r_hbm.at[pl.ds(base, B_PER_W)], ridx_v)
    pltpu.sync_copy(tt_hbm.at[pl.ds(base, B_PER_W)], ttidx_v)
    pltpu.sync_copy(t_hbm.at[pl.ds(base, B_PER_W)], tidx_v)

    cp0 = pltpu.async_copy(ent_hbm.at[hidx_v], eh_v, sem0)
    cp1 = pltpu.async_copy(rel_hbm.at[ridx_v], rr_v, sem1)
    cp2 = pltpu.async_copy(rel_hbm.at[ttidx_v], rtt_v, sem2)
    cp3 = pltpu.async_copy(ent_hbm.at[tidx_v], et_v, sem3)
    cp0.wait()
    cp1.wait()
    cp2.wait()
    cp3.wait()

    for i in range(B_PER_W):
        acc = jnp.zeros((16,), jnp.float32)
        for j in range(EMBED // 16):
            sl = pl.ds(j * 16, 16)
            e = eh_v[i, sl] + rr_v[i, sl] + rtt_v[i, sl] - et_v[i, sl]
            acc = acc + e * e
        sums_v[pl.ds(i * 16, 16)] = acc

    lane = lax.iota(jnp.int32, 16)
    for g in range(B_PER_W // 16):
        bidx = lane * 16 + (g * 256)
        tot = jnp.zeros((16,), jnp.float32)
        for k in range(16):
            tot = tot + plsc.load_gather(sums_v, [bidx + k])
        out_v[pl.ds(g * 16, 16)] = tot

    pltpu.sync_copy(out_v, out_hbm.at[pl.ds(base, B_PER_W)])


@jax.jit
def _ttranse_scores(entity_embedding, relation_embedding, h, r, tt, t):
    call = functools.partial(
        pl.kernel,
        out_type=jax.ShapeDtypeStruct((TOTAL,), jnp.float32),
        mesh=plsc.VectorSubcoreMesh(core_axis_name="c", subcore_axis_name="s"),
        compiler_params=pltpu.CompilerParams(
            needs_layout_passes=False, use_tc_tiling_on_sc=False),
        scratch_types=[
            pltpu.VMEM((B_PER_W,), jnp.int32),
            pltpu.VMEM((B_PER_W,), jnp.int32),
            pltpu.VMEM((B_PER_W,), jnp.int32),
            pltpu.VMEM((B_PER_W,), jnp.int32),
            pltpu.VMEM((B_PER_W, EMBED), jnp.float32),
            pltpu.VMEM((B_PER_W, EMBED), jnp.float32),
            pltpu.VMEM((B_PER_W, EMBED), jnp.float32),
            pltpu.VMEM((B_PER_W, EMBED), jnp.float32),
            pltpu.VMEM((B_PER_W * 16,), jnp.float32),
            pltpu.VMEM((B_PER_W,), jnp.float32),
            pltpu.SemaphoreType.DMA,
            pltpu.SemaphoreType.DMA,
            pltpu.SemaphoreType.DMA,
            pltpu.SemaphoreType.DMA,
        ],
    )(_score_body)
    return call(entity_embedding, relation_embedding, h, r, tt, t)


def kernel(batch, corrupt_batch, entity_embedding, relation_embedding):
    h = jnp.concatenate([batch[:, 0], corrupt_batch[:, 0]]).astype(jnp.int32)
    r = jnp.concatenate([batch[:, 1], corrupt_batch[:, 1]]).astype(jnp.int32)
    t = jnp.concatenate([batch[:, 2], corrupt_batch[:, 2]]).astype(jnp.int32)
    tt = jnp.concatenate([batch[:, 3], corrupt_batch[:, 3]]).astype(jnp.int32)
    out = _ttranse_scores(entity_embedding, relation_embedding, h, r, tt, t)
    n = batch.shape[0]
    return (out[:n], out[n:])


# speedup vs baseline: 3.9488x; 2.6679x over previous
import functools

import jax
import jax.numpy as jnp
from jax import lax
from jax.experimental import pallas as pl
from jax.experimental.pallas import tpu as pltpu
from jax.experimental.pallas import tpu_sc as plsc

EMBED = 64
TOTAL = 2048
NUM_CORES = 2
NUM_SUBCORES = 16
DIMS_PER_CORE = EMBED // NUM_CORES
DIMS_PER_WORKER = DIMS_PER_CORE // NUM_SUBCORES
NROW = 16
NCOL = TOTAL // NROW
ENTITIES = 100000


def _score_body(entT_hbm, relT_hbm, h_hbm, r_hbm, tt_hbm, t_hbm, out_hbm,
                hidx_v, ridx_v, ttidx_v, tidx_v,
                col_v, diff_v, acc_v, shared_s):
    c = lax.axis_index("c")
    s = lax.axis_index("s")

    pltpu.sync_copy(h_hbm, hidx_v)
    pltpu.sync_copy(r_hbm, ridx_v)
    pltpu.sync_copy(tt_hbm, ttidx_v)
    pltpu.sync_copy(t_hbm, tidx_v)

    for k in range(DIMS_PER_WORKER):
        d = c * DIMS_PER_CORE + s * DIMS_PER_WORKER + k

        pltpu.sync_copy(entT_hbm.at[d], col_v)

        def ent_row(row, _):
            for j in range(NCOL // 16):
                base = row * NCOL + j * 16
                hi = hidx_v[pl.ds(base, 16)]
                ti = tidx_v[pl.ds(base, 16)]
                eh = plsc.load_gather(col_v, [hi])
                et = plsc.load_gather(col_v, [ti])
                diff_v[row, pl.ds(j * 16, 16)] = eh - et
            return 0

        lax.fori_loop(0, NROW, ent_row, 0)

        pltpu.sync_copy(relT_hbm.at[d], col_v)

        def rel_row(row, _):
            for j in range(NCOL // 16):
                base = row * NCOL + j * 16
                ri = ridx_v[pl.ds(base, 16)]
                tti = ttidx_v[pl.ds(base, 16)]
                rr = plsc.load_gather(col_v, [ri])
                rtt = plsc.load_gather(col_v, [tti])
                sl = pl.ds(j * 16, 16)
                e = diff_v[row, sl] + rr + rtt
                if k == 0:
                    acc_v[row, sl] = e * e
                else:
                    acc_v[row, sl] = acc_v[row, sl] + e * e
            return 0

        lax.fori_loop(0, NROW, rel_row, 0)

    rows = lax.iota(jnp.int32, 16)

    @pl.when(s == 0)
    def _():
        pltpu.sync_copy(acc_v, shared_s)

    plsc.subcore_barrier()

    @pl.when(s != 0)
    def _():
        pltpu.sync_copy(acc_v, shared_s.at[rows], add=True)

    plsc.subcore_barrier()

    @pl.when(s == 0)
    def _():
        pltpu.sync_copy(shared_s, out_hbm.at[c])


@jax.jit
def _ttranse_scores(entT, relT, h, r, tt, t):
    call = functools.partial(
        pl.kernel,
        out_type=jax.ShapeDtypeStruct((NUM_CORES, NROW, NCOL), jnp.float32),
        mesh=plsc.VectorSubcoreMesh(core_axis_name="c", subcore_axis_name="s"),
        compiler_params=pltpu.CompilerParams(
            needs_layout_passes=False, use_tc_tiling_on_sc=True),
        scratch_types=[
            pltpu.VMEM((TOTAL,), jnp.int32),
            pltpu.VMEM((TOTAL,), jnp.int32),
            pltpu.VMEM((TOTAL,), jnp.int32),
            pltpu.VMEM((TOTAL,), jnp.int32),
            pltpu.VMEM((ENTITIES,), jnp.float32),
            pltpu.VMEM((NROW, NCOL), jnp.float32),
            pltpu.VMEM((NROW, NCOL), jnp.float32),
            pltpu.VMEM_SHARED((NROW, NCOL), jnp.float32),
        ],
    )(_score_body)
    return call(entT, relT, h, r, tt, t)


def kernel(batch, corrupt_batch, entity_embedding, relation_embedding):
    h = jnp.concatenate([batch[:, 0], corrupt_batch[:, 0]]).astype(jnp.int32)
    r = jnp.concatenate([batch[:, 1], corrupt_batch[:, 1]]).astype(jnp.int32)
    t = jnp.concatenate([batch[:, 2], corrupt_batch[:, 2]]).astype(jnp.int32)
    tt = jnp.concatenate([batch[:, 3], corrupt_batch[:, 3]]).astype(jnp.int32)
    out = _ttranse_scores(entity_embedding.T, relation_embedding.T, h, r, tt, t)
    total = (out[0] + out[1]).reshape(TOTAL)
    n = batch.shape[0]
    return (total[:n], total[n:])
